# Initial kernel scaffold; baseline (speedup 1.0000x reference)
import jax
import jax.numpy as jnp
from jax.experimental import pallas as pl


def kernel(x, fact, inp, msg_to, order, params, bias):
    raise NotImplementedError("write your pallas kernel here")


# speedup vs baseline: 1.3992x; 1.3992x over previous
import functools

import jax
import jax.numpy as jnp
from jax.experimental import pallas as pl
from jax.experimental.pallas import tpu as pltpu


def _masked_mm_kernel(mask_ref, fact0_ref, xw_ref, inp_ref, params_ref,
                      bias_ref, out_ref, *, num_ids, order_static):
    tf = out_ref.shape[0]
    fp = jnp.ones_like(inp_ref[0])
    for i in range(order_static):
        m = mask_ref[i]
        fp = fp * (inp_ref[i] * m + (1.0 - m))
    f0 = fact0_ref[:]
    npad = xw_ref.shape[0]
    iota_n = jax.lax.broadcasted_iota(jnp.int32, (tf, npad), 1)
    onehot_n = (iota_n == f0[:, None]).astype(jnp.float32)
    ids_f = jnp.sum(onehot_n * xw_ref[:][None, :], axis=1)
    ids_i = ids_f.astype(jnp.int32)
    ppad = bias_ref.shape[0]
    iota_p = jax.lax.broadcasted_iota(jnp.int32, (tf, ppad), 1)
    onehot_p = (iota_p == ids_i[:, None]).astype(jnp.float32)
    acc = jnp.dot(onehot_p, bias_ref[:], preferred_element_type=jnp.float32)

    def body(p, acc):
        m = (ids_i == p).astype(jnp.float32)
        w = params_ref[p]
        return acc + jnp.dot(fp * m[:, None], w,
                             preferred_element_type=jnp.float32)

    acc = jax.lax.fori_loop(0, num_ids, body, acc)
    out_ref[...] = acc


def kernel(x, fact, inp, msg_to, order, params, bias):
    num_ids, H, O = params.shape
    order_static, F, _ = inp.shape
    n_rows = x.shape[0]
    m_atoms = int(round(float(num_ids) ** 0.5))

    idx = jnp.arange(order_static)
    mask = ((idx < order) & (idx != msg_to)).astype(jnp.float32)

    xw = (x[:, 1] * m_atoms + x[:, 2]).astype(jnp.float32)
    npad = ((n_rows + 127) // 128) * 128
    xw_pad = jnp.zeros((npad,), jnp.float32).at[:n_rows].set(xw)
    fact0 = fact[:, 0].astype(jnp.int32)

    ppad = ((num_ids + 127) // 128) * 128
    bias_pad = jnp.zeros((ppad, O), jnp.float32).at[:num_ids].set(
        bias.reshape(num_ids, O))

    TF = 512
    grid = (F // TF,)
    out = pl.pallas_call(
        functools.partial(_masked_mm_kernel, num_ids=num_ids,
                          order_static=order_static),
        grid=grid,
        in_specs=[
            pl.BlockSpec(memory_space=pltpu.SMEM),
            pl.BlockSpec((TF,), lambda t: (t,)),
            pl.BlockSpec((npad,), lambda t: (0,)),
            pl.BlockSpec((order_static, TF, H), lambda t: (0, t, 0)),
            pl.BlockSpec((num_ids, H, O), lambda t: (0, 0, 0)),
            pl.BlockSpec((ppad, O), lambda t: (0, 0)),
        ],
        out_specs=pl.BlockSpec((TF, O), lambda t: (t, 0)),
        out_shape=jax.ShapeDtypeStruct((F, O), jnp.float32),
    )(mask, fact0, xw_pad, inp, params, bias_pad)
    return out
